# Initial kernel scaffold; baseline (speedup 1.0000x reference)
import jax
import jax.numpy as jnp
from jax.experimental import pallas as pl


def kernel(h, block0_edge_index, block1_edge_index, W1, b1, W2, b2, Wp, bp):
    raise NotImplementedError("write your pallas kernel here")


# speedup vs baseline: 10.5980x; 10.5980x over previous
import functools

import jax
import jax.numpy as jnp
from jax import lax
from jax.experimental import pallas as pl
from jax.experimental.pallas import tpu as pltpu
from jax.experimental.pallas import tpu_sc as plsc

N_N = 10000
N_E = 320000
D = 128
NC = 2
NS = 16
L = 128
N_DUMP = 8
ACC_N = N_N + N_DUMP
ROWS_TOTAL = 2560
PAD = ROWS_TOTAL * L - N_E
ROWS_PER_CORE = ROWS_TOTAL // NC
BULK_ROWS = ROWS_PER_CORE // NS
SLAB = 624
SLAB_REM = N_N - NS * SLAB
DEG_N = 10240
RB = 1024
NB = (N_N + RB - 1) // RB

_mesh = plsc.VectorSubcoreMesh(core_axis_name="c", subcore_axis_name="s")


@functools.partial(
    pl.kernel,
    mesh=_mesh,
    out_type=jax.ShapeDtypeStruct((NC * 4 * DEG_N,), jnp.float32),
    scratch_types=[
        pltpu.VMEM((BULK_ROWS, L), jnp.int32),
        pltpu.VMEM((1, L), jnp.float32),
        pltpu.VMEM((1024,), jnp.float32),
        pltpu.VMEM_SHARED((ACC_N,), jnp.float32),
        pltpu.VMEM_SHARED((ACC_N,), jnp.float32),
        pltpu.VMEM_SHARED((ACC_N,), jnp.float32),
        pltpu.VMEM_SHARED((ACC_N,), jnp.float32),
    ],
)
def _sc_degrees(s0, d0, s1, d1, out, idx_v, ones_v, zb_v, c0, c1, c2, c3):
    c = lax.axis_index("c")
    s = lax.axis_index("s")
    zeros16 = jnp.zeros((16,), jnp.float32)
    ones16 = jnp.ones((16,), jnp.float32)

    def _fill_z(i, carry):
        zb_v[pl.ds(i * 16, 16)] = zeros16
        return carry
    lax.fori_loop(0, 1024 // 16, _fill_z, 0)

    def _fill_o(i, carry):
        ones_v[0, pl.ds(i * 16, 16)] = ones16
        return carry
    lax.fori_loop(0, L // 16, _fill_o, 0)

    @pl.when(s < 10)
    def _():
        for cref in (c0, c1, c2, c3):
            pltpu.sync_copy(zb_v.at[pl.ds(0, 1000)],
                            cref.at[pl.ds(s * 1000, 1000)])
    plsc.subcore_barrier()

    for arr, cref in ((s0, c0), (d0, c1), (s1, c2), (d1, c3)):
        rbase = c * ROWS_PER_CORE + s * BULK_ROWS
        pltpu.sync_copy(arr.at[pl.ds(rbase, BULK_ROWS)], idx_v)

        def _cnt(j, carry):
            pltpu.sync_copy(ones_v.at[0], cref.at[idx_v.at[j]], add=True)
            return carry
        lax.fori_loop(0, BULK_ROWS, _cnt, 0)

    plsc.subcore_barrier()

    @pl.when(s < 10)
    def _():
        for a, cref in enumerate((c0, c1, c2, c3)):
            pltpu.sync_copy(cref.at[pl.ds(s * 1000, 1000)],
                            zb_v.at[pl.ds(0, 1000)])
            pltpu.sync_copy(
                zb_v.at[pl.ds(0, 1000)],
                out.at[pl.ds((c * 4 + a) * DEG_N + s * 1000, 1000)])


@functools.partial(
    pl.kernel,
    mesh=_mesh,
    out_type=jax.ShapeDtypeStruct((NC, N_N, D), jnp.float32),
    scratch_types=[
        pltpu.VMEM((BULK_ROWS // 2, L), jnp.int32),
        pltpu.VMEM((BULK_ROWS // 2, L), jnp.int32),
        pltpu.VMEM((L, D), jnp.float32),
        pltpu.VMEM((L, D), jnp.float32),
        pltpu.VMEM_SHARED((ACC_N, D), jnp.float32),
        pltpu.SemaphoreType.DMA,
        pltpu.SemaphoreType.DMA,
    ],
)
def _sc_message_pass(y, src, dst, out, sidx, didx, rows0, rows1,
                     acc, sem0, sem1):
    c = lax.axis_index("c")
    s = lax.axis_index("s")
    zeros16 = jnp.zeros((16,), jnp.float32)

    def _zo(i, carry):
        def _zi(k, carry2):
            rows0[i, pl.ds(k * 16, 16)] = zeros16
            return carry2
        return lax.fori_loop(0, D // 16, _zi, carry)
    lax.fori_loop(0, L, _zo, 0)

    slab = s * SLAB
    for off, n in ((0, 128), (128, 128), (256, 128), (384, 128), (512, 112)):
        pltpu.sync_copy(rows0.at[pl.ds(0, n)], acc.at[pl.ds(slab + off, n)])

    @pl.when(s == 0)
    def _():
        pltpu.sync_copy(rows0.at[pl.ds(0, SLAB_REM)],
                        acc.at[pl.ds(NS * SLAB, SLAB_REM)])

    plsc.subcore_barrier()

    HALF = BULK_ROWS // 2
    rbase = c * ROWS_PER_CORE + s * BULK_ROWS
    for phase in range(2):
        pbase = rbase + phase * HALF
        pltpu.sync_copy(src.at[pl.ds(pbase, HALF)], sidx)
        pltpu.sync_copy(dst.at[pl.ds(pbase, HALF)], didx)

        pltpu.make_async_copy(y.at[sidx.at[0]], rows0, sem0).start()

        def _step(it, carry):
            j = it * 2
            pltpu.make_async_copy(y.at[sidx.at[j + 1]], rows1, sem1).start()
            pltpu.make_async_copy(y.at[sidx.at[j]], rows0, sem0).wait()
            pltpu.sync_copy(rows0, acc.at[didx.at[j]], add=True)

            @pl.when(it < HALF // 2 - 1)
            def _():
                pltpu.make_async_copy(y.at[sidx.at[j + 2]], rows0, sem0).start()
            pltpu.make_async_copy(y.at[sidx.at[j + 1]], rows1, sem1).wait()
            pltpu.sync_copy(rows1, acc.at[didx.at[j + 1]], add=True)
            return carry
        lax.fori_loop(0, HALF // 2, _step, 0)

    plsc.subcore_barrier()
    for off, n in ((0, 128), (128, 128), (256, 128), (384, 128), (512, 112)):
        pltpu.sync_copy(acc.at[pl.ds(slab + off, n)], rows1.at[pl.ds(0, n)])
        pltpu.sync_copy(rows1.at[pl.ds(0, n)], out.at[c, pl.ds(slab + off, n)])

    @pl.when(s == 0)
    def _():
        pltpu.sync_copy(acc.at[pl.ds(NS * SLAB, SLAB_REM)],
                        rows0.at[pl.ds(0, SLAB_REM)])
        pltpu.sync_copy(rows0.at[pl.ds(0, SLAB_REM)],
                        out.at[c, pl.ds(NS * SLAB, SLAB_REM)])


def _tc_first_body(cnt_ref, h_ref, w_ref, o_ref):
    i = pl.program_id(0)
    sl = pl.ds(i * RB, RB)
    deg = cnt_ref[0, 0, sl] + cnt_ref[1, 0, sl]
    ns = lax.rsqrt(jnp.maximum(deg, 1.0))
    o_ref[...] = jnp.dot(h_ref[...] * ns[:, None], w_ref[...],
                         preferred_element_type=jnp.float32)


def _tc_mid_body(cnt_ref, p_ref, b_ref, w_ref, o_ref):
    i = pl.program_id(0)
    sl = pl.ds(i * RB, RB)
    din = cnt_ref[0, 1, sl] + cnt_ref[1, 1, sl]
    dout = cnt_ref[0, 2, sl] + cnt_ref[1, 2, sl]
    nd = lax.rsqrt(jnp.maximum(din, 1.0))
    ns = lax.rsqrt(jnp.maximum(dout, 1.0))
    agg = (p_ref[0, :, :] + p_ref[1, :, :]) * nd[:, None] + b_ref[...]
    o_ref[...] = jnp.dot(agg * ns[:, None], w_ref[...],
                         preferred_element_type=jnp.float32)


def _tc_last_body(cnt_ref, p_ref, b_ref, w_ref, bp_ref, o_ref):
    i = pl.program_id(0)
    sl = pl.ds(i * RB, RB)
    din = cnt_ref[0, 3, sl] + cnt_ref[1, 3, sl]
    nd = lax.rsqrt(jnp.maximum(din, 1.0))
    agg = (p_ref[0, :, :] + p_ref[1, :, :]) * nd[:, None] + b_ref[...]
    o_ref[...] = jnp.dot(agg, w_ref[...],
                         preferred_element_type=jnp.float32) + bp_ref[...]


_cnt_spec = pl.BlockSpec((NC, 4, DEG_N), lambda i: (0, 0, 0))
_row_spec = pl.BlockSpec((RB, D), lambda i: (i, 0))
_p_spec = pl.BlockSpec((NC, RB, D), lambda i: (0, i, 0))
_w_spec = pl.BlockSpec((D, D), lambda i: (0, 0))
_b_spec = pl.BlockSpec((1, D), lambda i: (0, 0))
_out_struct = jax.ShapeDtypeStruct((N_N, D), jnp.float32)

_tc_first = pl.pallas_call(
    _tc_first_body, grid=(NB,),
    in_specs=[_cnt_spec, _row_spec, _w_spec],
    out_specs=_row_spec, out_shape=_out_struct)

_tc_mid = pl.pallas_call(
    _tc_mid_body, grid=(NB,),
    in_specs=[_cnt_spec, _p_spec, _b_spec, _w_spec],
    out_specs=_row_spec, out_shape=_out_struct)

_tc_last = pl.pallas_call(
    _tc_last_body, grid=(NB,),
    in_specs=[_cnt_spec, _p_spec, _b_spec, _w_spec, _b_spec],
    out_specs=_row_spec, out_shape=_out_struct)


def kernel(h, block0_edge_index, block1_edge_index, W1, b1, W2, b2, Wp, bp):
    pad_i = jnp.arange(PAD, dtype=jnp.int32)
    pad_dump = (N_N + pad_i % N_DUMP).astype(jnp.int32)
    pad_inb = (pad_i % N_N).astype(jnp.int32)

    def _rows(a, pad):
        a = jnp.concatenate([a.astype(jnp.int32), pad])
        return a.reshape(ROWS_TOTAL, L)

    s0g = _rows(block0_edge_index[0], pad_inb)
    s0d = _rows(block0_edge_index[0], pad_dump)
    d0 = _rows(block0_edge_index[1], pad_dump)
    s1g = _rows(block1_edge_index[0], pad_inb)
    s1d = _rows(block1_edge_index[0], pad_dump)
    d1 = _rows(block1_edge_index[1], pad_dump)

    cnts = _sc_degrees(s0d, d0, s1d, d1).reshape(NC, 4, DEG_N)
    y0 = _tc_first(cnts, h, W1)
    p0 = _sc_message_pass(y0, s0g, d0)
    y1 = _tc_mid(cnts, p0, b1.reshape(1, D), W2)
    p1 = _sc_message_pass(y1, s1g, d1)
    out = _tc_last(cnts, p1, b2.reshape(1, D), Wp, bp.reshape(1, D))
    return out


# speedup vs baseline: 10.6851x; 1.0082x over previous
import functools

import jax
import jax.numpy as jnp
from jax import lax
from jax.experimental import pallas as pl
from jax.experimental.pallas import tpu as pltpu
from jax.experimental.pallas import tpu_sc as plsc

N_N = 10000
N_E = 320000
D = 128
NC = 2
NS = 16
L = 128
N_DUMP = 8
ACC_N = N_N + N_DUMP
ROWS_TOTAL = 2560
PAD = ROWS_TOTAL * L - N_E
ROWS_PER_CORE = ROWS_TOTAL // NC
BULK_ROWS = ROWS_PER_CORE // NS
SLAB = 624
SLAB_REM = N_N - NS * SLAB
DEG_N = 10240
RB = 1024
NB = (N_N + RB - 1) // RB

_mesh = plsc.VectorSubcoreMesh(core_axis_name="c", subcore_axis_name="s")


@functools.partial(
    pl.kernel,
    mesh=_mesh,
    out_type=jax.ShapeDtypeStruct((NC * 4 * DEG_N,), jnp.float32),
    scratch_types=[
        pltpu.VMEM((BULK_ROWS, L), jnp.int32),
        pltpu.VMEM((1, L), jnp.float32),
        pltpu.VMEM((1024,), jnp.float32),
        pltpu.VMEM_SHARED((ACC_N,), jnp.float32),
        pltpu.VMEM_SHARED((ACC_N,), jnp.float32),
        pltpu.VMEM_SHARED((ACC_N,), jnp.float32),
        pltpu.VMEM_SHARED((ACC_N,), jnp.float32),
        pltpu.SemaphoreType.DMA,
    ],
)
def _sc_degrees(s0, d0, s1, d1, out, idx_v, ones_v, zb_v, c0, c1, c2, c3,
                sem):
    c = lax.axis_index("c")
    s = lax.axis_index("s")
    zeros16 = jnp.zeros((16,), jnp.float32)
    ones16 = jnp.ones((16,), jnp.float32)

    def _fill_z(i, carry):
        zb_v[pl.ds(i * 16, 16)] = zeros16
        return carry
    lax.fori_loop(0, 1024 // 16, _fill_z, 0)

    def _fill_o(i, carry):
        ones_v[0, pl.ds(i * 16, 16)] = ones16
        return carry
    lax.fori_loop(0, L // 16, _fill_o, 0)

    @pl.when(s < 10)
    def _():
        for cref in (c0, c1, c2, c3):
            pltpu.sync_copy(zb_v.at[pl.ds(0, 1000)],
                            cref.at[pl.ds(s * 1000, 1000)])
    plsc.subcore_barrier()

    for arr, cref in ((s0, c0), (d0, c1), (s1, c2), (d1, c3)):
        rbase = c * ROWS_PER_CORE + s * BULK_ROWS
        pltpu.sync_copy(arr.at[pl.ds(rbase, BULK_ROWS)], idx_v)

        def _cnt(j, carry):
            pltpu.sync_copy(ones_v.at[0], cref.at[idx_v.at[j]], add=True)
            return carry
        lax.fori_loop(0, BULK_ROWS, _cnt, 0)

    plsc.subcore_barrier()

    @pl.when(s < 10)
    def _():
        for a, cref in enumerate((c0, c1, c2, c3)):
            pltpu.sync_copy(cref.at[pl.ds(s * 1000, 1000)],
                            zb_v.at[pl.ds(0, 1000)])
            pltpu.sync_copy(
                zb_v.at[pl.ds(0, 1000)],
                out.at[pl.ds((c * 4 + a) * DEG_N + s * 1000, 1000)])


@functools.partial(
    pl.kernel,
    mesh=_mesh,
    out_type=jax.ShapeDtypeStruct((NC, N_N, D), jnp.float32),
    scratch_types=[
        pltpu.VMEM((BULK_ROWS // 2, L), jnp.int32),
        pltpu.VMEM((BULK_ROWS // 2, L), jnp.int32),
        pltpu.VMEM((L, D), jnp.float32),
        pltpu.VMEM((L, D), jnp.float32),
        pltpu.VMEM_SHARED((ACC_N, D), jnp.float32),
        pltpu.SemaphoreType.DMA,
        pltpu.SemaphoreType.DMA,
        pltpu.SemaphoreType.DMA,
        pltpu.SemaphoreType.DMA,
    ],
)
def _sc_message_pass(y, src, dst, out, sidx, didx, rows0, rows1,
                     acc, gsem0, gsem1, ssem0, ssem1):
    c = lax.axis_index("c")
    s = lax.axis_index("s")
    zeros16 = jnp.zeros((16,), jnp.float32)

    def _zo(i, carry):
        def _zi(k, carry2):
            rows0[i, pl.ds(k * 16, 16)] = zeros16
            return carry2
        return lax.fori_loop(0, D // 16, _zi, carry)
    lax.fori_loop(0, L, _zo, 0)

    slab = s * SLAB
    for off, n in ((0, 128), (128, 128), (256, 128), (384, 128), (512, 112)):
        pltpu.sync_copy(rows0.at[pl.ds(0, n)], acc.at[pl.ds(slab + off, n)])

    @pl.when(s == 0)
    def _():
        pltpu.sync_copy(rows0.at[pl.ds(0, SLAB_REM)],
                        acc.at[pl.ds(NS * SLAB, SLAB_REM)])

    plsc.subcore_barrier()

    HALF = BULK_ROWS // 2
    rbase = c * ROWS_PER_CORE + s * BULK_ROWS
    for phase in range(2):
        pbase = rbase + phase * HALF
        pltpu.sync_copy(src.at[pl.ds(pbase, HALF)], sidx)
        pltpu.sync_copy(dst.at[pl.ds(pbase, HALF)], didx)

        pltpu.async_copy(y.at[sidx.at[0]], rows0, gsem0)

        def _step(it, carry):
            j = it * 2
            @pl.when(it > 0)
            def _():
                pltpu.make_async_copy(rows1, acc.at[didx.at[0]], ssem1).wait()
            pltpu.async_copy(y.at[sidx.at[j + 1]], rows1, gsem1)
            pltpu.make_async_copy(y.at[sidx.at[0]], rows0, gsem0).wait()
            pltpu.async_copy(rows0, acc.at[didx.at[j]], ssem0, add=True)
            pltpu.make_async_copy(rows0, acc.at[didx.at[0]], ssem0).wait()

            @pl.when(it < HALF // 2 - 1)
            def _():
                pltpu.async_copy(y.at[sidx.at[j + 2]], rows0, gsem0)
            pltpu.make_async_copy(y.at[sidx.at[0]], rows1, gsem1).wait()
            pltpu.async_copy(rows1, acc.at[didx.at[j + 1]], ssem1, add=True)
            return carry
        lax.fori_loop(0, HALF // 2, _step, 0)
        pltpu.make_async_copy(rows1, acc.at[didx.at[0]], ssem1).wait()

    plsc.subcore_barrier()
    chunks = ((0, 128), (128, 128), (256, 128), (384, 128), (512, 112))
    bufs, gs, ss = (rows0, rows1), (gsem0, gsem1), (ssem0, ssem1)

    def _cin(k, b):
        off, n = chunks[k]
        pltpu.async_copy(acc.at[pl.ds(slab + off, n)],
                         bufs[b].at[pl.ds(0, n)], gs[b])

    _cin(0, 0)
    for k in range(5):
        off, n = chunks[k]
        b = k % 2
        pltpu.make_async_copy(acc.at[pl.ds(slab + off, n)],
                              bufs[b].at[pl.ds(0, n)], gs[b]).wait()
        if k + 1 < 5:
            nb = (k + 1) % 2
            if k + 1 >= 2:
                poff, pn = chunks[k - 1]
                pltpu.make_async_copy(
                    bufs[nb].at[pl.ds(0, pn)],
                    out.at[c, pl.ds(slab + poff, pn)], ss[nb]).wait()
            _cin(k + 1, nb)
        pltpu.async_copy(bufs[b].at[pl.ds(0, n)],
                         out.at[c, pl.ds(slab + off, n)], ss[b])
    for k in (3, 4):
        off, n = chunks[k]
        pltpu.make_async_copy(bufs[k % 2].at[pl.ds(0, n)],
                              out.at[c, pl.ds(slab + off, n)],
                              ss[k % 2]).wait()

    @pl.when(s == 0)
    def _():
        pltpu.sync_copy(acc.at[pl.ds(NS * SLAB, SLAB_REM)],
                        rows0.at[pl.ds(0, SLAB_REM)])
        pltpu.sync_copy(rows0.at[pl.ds(0, SLAB_REM)],
                        out.at[c, pl.ds(NS * SLAB, SLAB_REM)])


def _tc_first_body(cnt_ref, h_ref, w_ref, o_ref):
    i = pl.program_id(0)
    sl = pl.ds(i * RB, RB)
    deg = cnt_ref[0, 0, sl] + cnt_ref[1, 0, sl]
    ns = lax.rsqrt(jnp.maximum(deg, 1.0))
    o_ref[...] = jnp.dot(h_ref[...] * ns[:, None], w_ref[...],
                         preferred_element_type=jnp.float32)


def _tc_mid_body(cnt_ref, p_ref, b_ref, w_ref, o_ref):
    i = pl.program_id(0)
    sl = pl.ds(i * RB, RB)
    din = cnt_ref[0, 1, sl] + cnt_ref[1, 1, sl]
    dout = cnt_ref[0, 2, sl] + cnt_ref[1, 2, sl]
    nd = lax.rsqrt(jnp.maximum(din, 1.0))
    ns = lax.rsqrt(jnp.maximum(dout, 1.0))
    agg = (p_ref[0, :, :] + p_ref[1, :, :]) * nd[:, None] + b_ref[...]
    o_ref[...] = jnp.dot(agg * ns[:, None], w_ref[...],
                         preferred_element_type=jnp.float32)


def _tc_last_body(cnt_ref, p_ref, b_ref, w_ref, bp_ref, o_ref):
    i = pl.program_id(0)
    sl = pl.ds(i * RB, RB)
    din = cnt_ref[0, 3, sl] + cnt_ref[1, 3, sl]
    nd = lax.rsqrt(jnp.maximum(din, 1.0))
    agg = (p_ref[0, :, :] + p_ref[1, :, :]) * nd[:, None] + b_ref[...]
    o_ref[...] = jnp.dot(agg, w_ref[...],
                         preferred_element_type=jnp.float32) + bp_ref[...]


_cnt_spec = pl.BlockSpec((NC, 4, DEG_N), lambda i: (0, 0, 0))
_row_spec = pl.BlockSpec((RB, D), lambda i: (i, 0))
_p_spec = pl.BlockSpec((NC, RB, D), lambda i: (0, i, 0))
_w_spec = pl.BlockSpec((D, D), lambda i: (0, 0))
_b_spec = pl.BlockSpec((1, D), lambda i: (0, 0))
_out_struct = jax.ShapeDtypeStruct((N_N, D), jnp.float32)

_tc_first = pl.pallas_call(
    _tc_first_body, grid=(NB,),
    in_specs=[_cnt_spec, _row_spec, _w_spec],
    out_specs=_row_spec, out_shape=_out_struct)

_tc_mid = pl.pallas_call(
    _tc_mid_body, grid=(NB,),
    in_specs=[_cnt_spec, _p_spec, _b_spec, _w_spec],
    out_specs=_row_spec, out_shape=_out_struct)

_tc_last = pl.pallas_call(
    _tc_last_body, grid=(NB,),
    in_specs=[_cnt_spec, _p_spec, _b_spec, _w_spec, _b_spec],
    out_specs=_row_spec, out_shape=_out_struct)


def kernel(h, block0_edge_index, block1_edge_index, W1, b1, W2, b2, Wp, bp):
    pad_i = jnp.arange(PAD, dtype=jnp.int32)
    pad_dump = (N_N + pad_i % N_DUMP).astype(jnp.int32)
    pad_inb = (pad_i % N_N).astype(jnp.int32)

    def _rows(a, pad):
        a = jnp.concatenate([a.astype(jnp.int32), pad])
        return a.reshape(ROWS_TOTAL, L)

    s0g = _rows(block0_edge_index[0], pad_inb)
    s0d = _rows(block0_edge_index[0], pad_dump)
    d0 = _rows(block0_edge_index[1], pad_dump)
    s1g = _rows(block1_edge_index[0], pad_inb)
    s1d = _rows(block1_edge_index[0], pad_dump)
    d1 = _rows(block1_edge_index[1], pad_dump)

    cnts = _sc_degrees(s0d, d0, s1d, d1).reshape(NC, 4, DEG_N)
    y0 = _tc_first(cnts, h, W1)
    p0 = _sc_message_pass(y0, s0g, d0)
    y1 = _tc_mid(cnts, p0, b1.reshape(1, D), W2)
    p1 = _sc_message_pass(y1, s1g, d1)
    out = _tc_last(cnts, p1, b2.reshape(1, D), Wp, bp.reshape(1, D))
    return out


# speedup vs baseline: 11.0469x; 1.0339x over previous
import functools

import jax
import jax.numpy as jnp
from jax import lax
from jax.experimental import pallas as pl
from jax.experimental.pallas import tpu as pltpu
from jax.experimental.pallas import tpu_sc as plsc

N_N = 10000
N_E = 320000
D = 128
NC = 2
NS = 16
L = 128
N_DUMP = 8
ACC_N = N_N + N_DUMP
ROWS_TOTAL = 2560
PAD = ROWS_TOTAL * L - N_E
ROWS_PER_CORE = ROWS_TOTAL // NC
BULK_ROWS = ROWS_PER_CORE // NS
SLAB = 624
SLAB_REM = N_N - NS * SLAB
DEG_N = 10240
RB = 1024
NB = (N_N + RB - 1) // RB

_mesh = plsc.VectorSubcoreMesh(core_axis_name="c", subcore_axis_name="s")


@functools.partial(
    pl.kernel,
    mesh=_mesh,
    out_type=jax.ShapeDtypeStruct((NC * 4 * DEG_N,), jnp.float32),
    scratch_types=[
        pltpu.VMEM((BULK_ROWS, L), jnp.int32),
        pltpu.VMEM((BULK_ROWS, L), jnp.int32),
        pltpu.VMEM((BULK_ROWS, L), jnp.int32),
        pltpu.VMEM((BULK_ROWS, L), jnp.int32),
        pltpu.VMEM((1, L), jnp.float32),
        pltpu.VMEM((1024,), jnp.float32),
        pltpu.VMEM_SHARED((ACC_N,), jnp.float32),
        pltpu.VMEM_SHARED((ACC_N,), jnp.float32),
        pltpu.VMEM_SHARED((ACC_N,), jnp.float32),
        pltpu.VMEM_SHARED((ACC_N,), jnp.float32),
        pltpu.SemaphoreType.DMA,
        pltpu.SemaphoreType.DMA,
        pltpu.SemaphoreType.DMA,
        pltpu.SemaphoreType.DMA,
    ],
)
def _sc_degrees(s0, d0, s1, d1, out, i0, i1, i2, i3, ones_v, zb_v,
                c0, c1, c2, c3, m0, m1, m2, m3):
    c = lax.axis_index("c")
    s = lax.axis_index("s")
    zeros16 = jnp.zeros((16,), jnp.float32)
    ones16 = jnp.ones((16,), jnp.float32)

    def _fill_z(i, carry):
        zb_v[pl.ds(i * 16, 16)] = zeros16
        return carry
    lax.fori_loop(0, 1024 // 16, _fill_z, 0)

    def _fill_o(i, carry):
        ones_v[0, pl.ds(i * 16, 16)] = ones16
        return carry
    lax.fori_loop(0, L // 16, _fill_o, 0)

    @pl.when(s < 10)
    def _():
        for cref in (c0, c1, c2, c3):
            pltpu.sync_copy(zb_v.at[pl.ds(0, 1000)],
                            cref.at[pl.ds(s * 1000, 1000)])
    plsc.subcore_barrier()

    rbase = c * ROWS_PER_CORE + s * BULK_ROWS
    quad = ((s0, i0, c0, m0), (d0, i1, c1, m1),
            (s1, i2, c2, m2), (d1, i3, c3, m3))
    for arr, iv, cref, m in quad:
        pltpu.sync_copy(arr.at[pl.ds(rbase, BULK_ROWS)], iv)

    def _cnt(j, carry):
        for arr, iv, cref, m in quad:
            pltpu.async_copy(ones_v.at[0], cref.at[iv.at[j]], m, add=True)
        for arr, iv, cref, m in quad:
            pltpu.make_async_copy(ones_v.at[0], cref.at[iv.at[0]], m).wait()
        return carry
    lax.fori_loop(0, BULK_ROWS, _cnt, 0)

    plsc.subcore_barrier()

    @pl.when(s < 10)
    def _():
        for a, cref in enumerate((c0, c1, c2, c3)):
            pltpu.sync_copy(cref.at[pl.ds(s * 1000, 1000)],
                            zb_v.at[pl.ds(0, 1000)])
            pltpu.sync_copy(
                zb_v.at[pl.ds(0, 1000)],
                out.at[pl.ds((c * 4 + a) * DEG_N + s * 1000, 1000)])


@functools.partial(
    pl.kernel,
    mesh=_mesh,
    out_type=jax.ShapeDtypeStruct((NC, N_N, D), jnp.float32),
    scratch_types=[
        pltpu.VMEM((BULK_ROWS // 2, L), jnp.int32),
        pltpu.VMEM((BULK_ROWS // 2, L), jnp.int32),
        pltpu.VMEM((L, D), jnp.float32),
        pltpu.VMEM((L, D), jnp.float32),
        pltpu.VMEM_SHARED((ACC_N, D), jnp.float32),
        pltpu.SemaphoreType.DMA,
        pltpu.SemaphoreType.DMA,
        pltpu.SemaphoreType.DMA,
        pltpu.SemaphoreType.DMA,
    ],
)
def _sc_message_pass(y, src, dst, out, sidx, didx, rows0, rows1,
                     acc, gsem0, gsem1, ssem0, ssem1):
    c = lax.axis_index("c")
    s = lax.axis_index("s")
    zeros16 = jnp.zeros((16,), jnp.float32)

    def _zo(i, carry):
        def _zi(k, carry2):
            rows0[i, pl.ds(k * 16, 16)] = zeros16
            return carry2
        return lax.fori_loop(0, D // 16, _zi, carry)
    lax.fori_loop(0, L, _zo, 0)

    slab = s * SLAB
    for off, n in ((0, 128), (128, 128), (256, 128), (384, 128), (512, 112)):
        pltpu.sync_copy(rows0.at[pl.ds(0, n)], acc.at[pl.ds(slab + off, n)])

    @pl.when(s == 0)
    def _():
        pltpu.sync_copy(rows0.at[pl.ds(0, SLAB_REM)],
                        acc.at[pl.ds(NS * SLAB, SLAB_REM)])

    plsc.subcore_barrier()

    HALF = BULK_ROWS // 2
    rbase = c * ROWS_PER_CORE + s * BULK_ROWS
    for phase in range(2):
        pbase = rbase + phase * HALF
        pltpu.sync_copy(src.at[pl.ds(pbase, HALF)], sidx)
        pltpu.sync_copy(dst.at[pl.ds(pbase, HALF)], didx)

        pltpu.async_copy(y.at[sidx.at[0]], rows0, gsem0)

        def _step(it, carry):
            j = it * 2
            @pl.when(it > 0)
            def _():
                pltpu.make_async_copy(rows1, acc.at[didx.at[0]], ssem1).wait()
            pltpu.async_copy(y.at[sidx.at[j + 1]], rows1, gsem1)
            pltpu.make_async_copy(y.at[sidx.at[0]], rows0, gsem0).wait()
            pltpu.async_copy(rows0, acc.at[didx.at[j]], ssem0, add=True)
            pltpu.make_async_copy(rows0, acc.at[didx.at[0]], ssem0).wait()

            @pl.when(it < HALF // 2 - 1)
            def _():
                pltpu.async_copy(y.at[sidx.at[j + 2]], rows0, gsem0)
            pltpu.make_async_copy(y.at[sidx.at[0]], rows1, gsem1).wait()
            pltpu.async_copy(rows1, acc.at[didx.at[j + 1]], ssem1, add=True)
            return carry
        lax.fori_loop(0, HALF // 2, _step, 0)
        pltpu.make_async_copy(rows1, acc.at[didx.at[0]], ssem1).wait()

    plsc.subcore_barrier()
    chunks = ((0, 128), (128, 128), (256, 128), (384, 128), (512, 112))
    bufs, gs, ss = (rows0, rows1), (gsem0, gsem1), (ssem0, ssem1)

    def _cin(k, b):
        off, n = chunks[k]
        pltpu.async_copy(acc.at[pl.ds(slab + off, n)],
                         bufs[b].at[pl.ds(0, n)], gs[b])

    _cin(0, 0)
    for k in range(5):
        off, n = chunks[k]
        b = k % 2
        pltpu.make_async_copy(acc.at[pl.ds(slab + off, n)],
                              bufs[b].at[pl.ds(0, n)], gs[b]).wait()
        if k + 1 < 5:
            nb = (k + 1) % 2
            if k + 1 >= 2:
                poff, pn = chunks[k - 1]
                pltpu.make_async_copy(
                    bufs[nb].at[pl.ds(0, pn)],
                    out.at[c, pl.ds(slab + poff, pn)], ss[nb]).wait()
            _cin(k + 1, nb)
        pltpu.async_copy(bufs[b].at[pl.ds(0, n)],
                         out.at[c, pl.ds(slab + off, n)], ss[b])
    for k in (3, 4):
        off, n = chunks[k]
        pltpu.make_async_copy(bufs[k % 2].at[pl.ds(0, n)],
                              out.at[c, pl.ds(slab + off, n)],
                              ss[k % 2]).wait()

    @pl.when(s == 0)
    def _():
        pltpu.sync_copy(acc.at[pl.ds(NS * SLAB, SLAB_REM)],
                        rows0.at[pl.ds(0, SLAB_REM)])
        pltpu.sync_copy(rows0.at[pl.ds(0, SLAB_REM)],
                        out.at[c, pl.ds(NS * SLAB, SLAB_REM)])


def _tc_first_body(cnt_ref, h_ref, w_ref, o_ref):
    i = pl.program_id(0)
    sl = pl.ds(i * RB, RB)
    deg = cnt_ref[0, 0, sl] + cnt_ref[1, 0, sl]
    ns = lax.rsqrt(jnp.maximum(deg, 1.0))
    o_ref[...] = jnp.dot(h_ref[...] * ns[:, None], w_ref[...],
                         preferred_element_type=jnp.float32)


def _tc_mid_body(cnt_ref, p_ref, b_ref, w_ref, o_ref):
    i = pl.program_id(0)
    sl = pl.ds(i * RB, RB)
    din = cnt_ref[0, 1, sl] + cnt_ref[1, 1, sl]
    dout = cnt_ref[0, 2, sl] + cnt_ref[1, 2, sl]
    nd = lax.rsqrt(jnp.maximum(din, 1.0))
    ns = lax.rsqrt(jnp.maximum(dout, 1.0))
    agg = (p_ref[0, :, :] + p_ref[1, :, :]) * nd[:, None] + b_ref[...]
    o_ref[...] = jnp.dot(agg * ns[:, None], w_ref[...],
                         preferred_element_type=jnp.float32)


def _tc_last_body(cnt_ref, p_ref, b_ref, w_ref, bp_ref, o_ref):
    i = pl.program_id(0)
    sl = pl.ds(i * RB, RB)
    din = cnt_ref[0, 3, sl] + cnt_ref[1, 3, sl]
    nd = lax.rsqrt(jnp.maximum(din, 1.0))
    agg = (p_ref[0, :, :] + p_ref[1, :, :]) * nd[:, None] + b_ref[...]
    o_ref[...] = jnp.dot(agg, w_ref[...],
                         preferred_element_type=jnp.float32) + bp_ref[...]


_cnt_spec = pl.BlockSpec((NC, 4, DEG_N), lambda i: (0, 0, 0))
_row_spec = pl.BlockSpec((RB, D), lambda i: (i, 0))
_p_spec = pl.BlockSpec((NC, RB, D), lambda i: (0, i, 0))
_w_spec = pl.BlockSpec((D, D), lambda i: (0, 0))
_b_spec = pl.BlockSpec((1, D), lambda i: (0, 0))
_out_struct = jax.ShapeDtypeStruct((N_N, D), jnp.float32)

_tc_first = pl.pallas_call(
    _tc_first_body, grid=(NB,),
    in_specs=[_cnt_spec, _row_spec, _w_spec],
    out_specs=_row_spec, out_shape=_out_struct)

_tc_mid = pl.pallas_call(
    _tc_mid_body, grid=(NB,),
    in_specs=[_cnt_spec, _p_spec, _b_spec, _w_spec],
    out_specs=_row_spec, out_shape=_out_struct)

_tc_last = pl.pallas_call(
    _tc_last_body, grid=(NB,),
    in_specs=[_cnt_spec, _p_spec, _b_spec, _w_spec, _b_spec],
    out_specs=_row_spec, out_shape=_out_struct)


def kernel(h, block0_edge_index, block1_edge_index, W1, b1, W2, b2, Wp, bp):
    pad_i = jnp.arange(PAD, dtype=jnp.int32)
    pad_dump = (N_N + pad_i % N_DUMP).astype(jnp.int32)
    pad_inb = (pad_i % N_N).astype(jnp.int32)

    def _rows(a, pad):
        a = jnp.concatenate([a.astype(jnp.int32), pad])
        return a.reshape(ROWS_TOTAL, L)

    s0g = _rows(block0_edge_index[0], pad_inb)
    s0d = _rows(block0_edge_index[0], pad_dump)
    d0 = _rows(block0_edge_index[1], pad_dump)
    s1g = _rows(block1_edge_index[0], pad_inb)
    s1d = _rows(block1_edge_index[0], pad_dump)
    d1 = _rows(block1_edge_index[1], pad_dump)

    cnts = _sc_degrees(s0d, d0, s1d, d1).reshape(NC, 4, DEG_N)
    y0 = _tc_first(cnts, h, W1)
    p0 = _sc_message_pass(y0, s0g, d0)
    y1 = _tc_mid(cnts, p0, b1.reshape(1, D), W2)
    p1 = _sc_message_pass(y1, s1g, d1)
    out = _tc_last(cnts, p1, b2.reshape(1, D), Wp, bp.reshape(1, D))
    return out


# speedup vs baseline: 11.2048x; 1.0143x over previous
import functools

import jax
import jax.numpy as jnp
from jax import lax
from jax.experimental import pallas as pl
from jax.experimental.pallas import tpu as pltpu
from jax.experimental.pallas import tpu_sc as plsc

N_N = 10000
N_E = 320000
D = 128
NC = 2
NS = 16
L = 128
N_DUMP = 8
ACC_N = N_N + N_DUMP
ROWS_TOTAL = 2560
PAD = ROWS_TOTAL * L - N_E
ROWS_PER_CORE = ROWS_TOTAL // NC
BULK_ROWS = ROWS_PER_CORE // NS
SLAB = 624
SLAB_REM = N_N - NS * SLAB
DEG_N = 10240
RB = 2048
NB = (N_N + RB - 1) // RB

_mesh = plsc.VectorSubcoreMesh(core_axis_name="c", subcore_axis_name="s")


@functools.partial(
    pl.kernel,
    mesh=_mesh,
    out_type=jax.ShapeDtypeStruct((NC * 4 * DEG_N,), jnp.float32),
    scratch_types=[
        pltpu.VMEM((BULK_ROWS, L), jnp.int32),
        pltpu.VMEM((BULK_ROWS, L), jnp.int32),
        pltpu.VMEM((BULK_ROWS, L), jnp.int32),
        pltpu.VMEM((BULK_ROWS, L), jnp.int32),
        pltpu.VMEM((1, L), jnp.float32),
        pltpu.VMEM((1024,), jnp.float32),
        pltpu.VMEM_SHARED((ACC_N,), jnp.float32),
        pltpu.VMEM_SHARED((ACC_N,), jnp.float32),
        pltpu.VMEM_SHARED((ACC_N,), jnp.float32),
        pltpu.VMEM_SHARED((ACC_N,), jnp.float32),
        pltpu.SemaphoreType.DMA,
        pltpu.SemaphoreType.DMA,
        pltpu.SemaphoreType.DMA,
        pltpu.SemaphoreType.DMA,
    ],
)
def _sc_degrees(s0, d0, s1, d1, out, i0, i1, i2, i3, ones_v, zb_v,
                c0, c1, c2, c3, m0, m1, m2, m3):
    c = lax.axis_index("c")
    s = lax.axis_index("s")
    zeros16 = jnp.zeros((16,), jnp.float32)
    ones16 = jnp.ones((16,), jnp.float32)

    def _fill_z(i, carry):
        zb_v[pl.ds(i * 16, 16)] = zeros16
        return carry
    lax.fori_loop(0, 1024 // 16, _fill_z, 0)

    def _fill_o(i, carry):
        ones_v[0, pl.ds(i * 16, 16)] = ones16
        return carry
    lax.fori_loop(0, L // 16, _fill_o, 0)

    @pl.when(s < 10)
    def _():
        for cref in (c0, c1, c2, c3):
            pltpu.sync_copy(zb_v.at[pl.ds(0, 1000)],
                            cref.at[pl.ds(s * 1000, 1000)])
    plsc.subcore_barrier()

    rbase = c * ROWS_PER_CORE + s * BULK_ROWS
    quad = ((s0, i0, c0, m0), (d0, i1, c1, m1),
            (s1, i2, c2, m2), (d1, i3, c3, m3))
    for arr, iv, cref, m in quad:
        pltpu.sync_copy(arr.at[pl.ds(rbase, BULK_ROWS)], iv)

    def _cnt(j, carry):
        for arr, iv, cref, m in quad:
            pltpu.async_copy(ones_v.at[0], cref.at[iv.at[j]], m, add=True)
        for arr, iv, cref, m in quad:
            pltpu.make_async_copy(ones_v.at[0], cref.at[iv.at[0]], m).wait()
        return carry
    lax.fori_loop(0, BULK_ROWS, _cnt, 0)

    plsc.subcore_barrier()

    @pl.when(s < 10)
    def _():
        for a, cref in enumerate((c0, c1, c2, c3)):
            pltpu.sync_copy(cref.at[pl.ds(s * 1000, 1000)],
                            zb_v.at[pl.ds(0, 1000)])
            pltpu.sync_copy(
                zb_v.at[pl.ds(0, 1000)],
                out.at[pl.ds((c * 4 + a) * DEG_N + s * 1000, 1000)])


@functools.partial(
    pl.kernel,
    mesh=_mesh,
    out_type=jax.ShapeDtypeStruct((NC, N_N, D), jnp.float32),
    scratch_types=[
        pltpu.VMEM((BULK_ROWS // 2, L), jnp.int32),
        pltpu.VMEM((BULK_ROWS // 2, L), jnp.int32),
        pltpu.VMEM((L, D), jnp.float32),
        pltpu.VMEM((L, D), jnp.float32),
        pltpu.VMEM_SHARED((ACC_N, D), jnp.float32),
        pltpu.SemaphoreType.DMA,
        pltpu.SemaphoreType.DMA,
        pltpu.SemaphoreType.DMA,
        pltpu.SemaphoreType.DMA,
    ],
)
def _sc_message_pass(y, src, dst, out, sidx, didx, rows0, rows1,
                     acc, gsem0, gsem1, ssem0, ssem1):
    c = lax.axis_index("c")
    s = lax.axis_index("s")
    zeros16 = jnp.zeros((16,), jnp.float32)

    def _zo(i, carry):
        def _zi(k, carry2):
            rows0[i, pl.ds(k * 16, 16)] = zeros16
            return carry2
        return lax.fori_loop(0, D // 16, _zi, carry)
    lax.fori_loop(0, L, _zo, 0)

    slab = s * SLAB
    for off, n in ((0, 128), (128, 128), (256, 128), (384, 128), (512, 112)):
        pltpu.sync_copy(rows0.at[pl.ds(0, n)], acc.at[pl.ds(slab + off, n)])

    @pl.when(s == 0)
    def _():
        pltpu.sync_copy(rows0.at[pl.ds(0, SLAB_REM)],
                        acc.at[pl.ds(NS * SLAB, SLAB_REM)])

    plsc.subcore_barrier()

    HALF = BULK_ROWS // 2
    rbase = c * ROWS_PER_CORE + s * BULK_ROWS
    for phase in range(2):
        pbase = rbase + phase * HALF
        pltpu.sync_copy(src.at[pl.ds(pbase, HALF)], sidx)
        pltpu.sync_copy(dst.at[pl.ds(pbase, HALF)], didx)

        pltpu.async_copy(y.at[sidx.at[0]], rows0, gsem0)

        def _step(it, carry):
            j = it * 2
            @pl.when(it > 0)
            def _():
                pltpu.make_async_copy(rows1, acc.at[didx.at[0]], ssem1).wait()
            pltpu.async_copy(y.at[sidx.at[j + 1]], rows1, gsem1)
            pltpu.make_async_copy(y.at[sidx.at[0]], rows0, gsem0).wait()
            pltpu.async_copy(rows0, acc.at[didx.at[j]], ssem0, add=True)
            pltpu.make_async_copy(rows0, acc.at[didx.at[0]], ssem0).wait()

            @pl.when(it < HALF // 2 - 1)
            def _():
                pltpu.async_copy(y.at[sidx.at[j + 2]], rows0, gsem0)
            pltpu.make_async_copy(y.at[sidx.at[0]], rows1, gsem1).wait()
            pltpu.async_copy(rows1, acc.at[didx.at[j + 1]], ssem1, add=True)
            return carry
        lax.fori_loop(0, HALF // 2, _step, 0)
        pltpu.make_async_copy(rows1, acc.at[didx.at[0]], ssem1).wait()

    plsc.subcore_barrier()
    chunks = ((0, 128), (128, 128), (256, 128), (384, 128), (512, 112))
    bufs, gs, ss = (rows0, rows1), (gsem0, gsem1), (ssem0, ssem1)

    def _cin(k, b):
        off, n = chunks[k]
        pltpu.async_copy(acc.at[pl.ds(slab + off, n)],
                         bufs[b].at[pl.ds(0, n)], gs[b])

    _cin(0, 0)
    for k in range(5):
        off, n = chunks[k]
        b = k % 2
        pltpu.make_async_copy(acc.at[pl.ds(slab + off, n)],
                              bufs[b].at[pl.ds(0, n)], gs[b]).wait()
        if k + 1 < 5:
            nb = (k + 1) % 2
            if k + 1 >= 2:
                poff, pn = chunks[k - 1]
                pltpu.make_async_copy(
                    bufs[nb].at[pl.ds(0, pn)],
                    out.at[c, pl.ds(slab + poff, pn)], ss[nb]).wait()
            _cin(k + 1, nb)
        pltpu.async_copy(bufs[b].at[pl.ds(0, n)],
                         out.at[c, pl.ds(slab + off, n)], ss[b])
    for k in (3, 4):
        off, n = chunks[k]
        pltpu.make_async_copy(bufs[k % 2].at[pl.ds(0, n)],
                              out.at[c, pl.ds(slab + off, n)],
                              ss[k % 2]).wait()

    @pl.when(s == 0)
    def _():
        pltpu.sync_copy(acc.at[pl.ds(NS * SLAB, SLAB_REM)],
                        rows0.at[pl.ds(0, SLAB_REM)])
        pltpu.sync_copy(rows0.at[pl.ds(0, SLAB_REM)],
                        out.at[c, pl.ds(NS * SLAB, SLAB_REM)])


def _tc_first_body(cnt_ref, h_ref, w_ref, o_ref):
    i = pl.program_id(0)
    sl = pl.ds(i * RB, RB)
    deg = cnt_ref[0, 0, sl] + cnt_ref[1, 0, sl]
    ns = lax.rsqrt(jnp.maximum(deg, 1.0))
    o_ref[...] = jnp.dot(h_ref[...] * ns[:, None], w_ref[...],
                         preferred_element_type=jnp.float32)


def _tc_mid_body(cnt_ref, p_ref, b_ref, w_ref, o_ref):
    i = pl.program_id(0)
    sl = pl.ds(i * RB, RB)
    din = cnt_ref[0, 1, sl] + cnt_ref[1, 1, sl]
    dout = cnt_ref[0, 2, sl] + cnt_ref[1, 2, sl]
    nd = lax.rsqrt(jnp.maximum(din, 1.0))
    ns = lax.rsqrt(jnp.maximum(dout, 1.0))
    agg = (p_ref[0, :, :] + p_ref[1, :, :]) * nd[:, None] + b_ref[...]
    o_ref[...] = jnp.dot(agg * ns[:, None], w_ref[...],
                         preferred_element_type=jnp.float32)


def _tc_last_body(cnt_ref, p_ref, b_ref, w_ref, bp_ref, o_ref):
    i = pl.program_id(0)
    sl = pl.ds(i * RB, RB)
    din = cnt_ref[0, 3, sl] + cnt_ref[1, 3, sl]
    nd = lax.rsqrt(jnp.maximum(din, 1.0))
    agg = (p_ref[0, :, :] + p_ref[1, :, :]) * nd[:, None] + b_ref[...]
    o_ref[...] = jnp.dot(agg, w_ref[...],
                         preferred_element_type=jnp.float32) + bp_ref[...]


_cnt_spec = pl.BlockSpec((NC, 4, DEG_N), lambda i: (0, 0, 0))
_row_spec = pl.BlockSpec((RB, D), lambda i: (i, 0))
_p_spec = pl.BlockSpec((NC, RB, D), lambda i: (0, i, 0))
_w_spec = pl.BlockSpec((D, D), lambda i: (0, 0))
_b_spec = pl.BlockSpec((1, D), lambda i: (0, 0))
_out_struct = jax.ShapeDtypeStruct((N_N, D), jnp.float32)

_tc_first = pl.pallas_call(
    _tc_first_body, grid=(NB,),
    in_specs=[_cnt_spec, _row_spec, _w_spec],
    out_specs=_row_spec, out_shape=_out_struct)

_tc_mid = pl.pallas_call(
    _tc_mid_body, grid=(NB,),
    in_specs=[_cnt_spec, _p_spec, _b_spec, _w_spec],
    out_specs=_row_spec, out_shape=_out_struct)

_tc_last = pl.pallas_call(
    _tc_last_body, grid=(NB,),
    in_specs=[_cnt_spec, _p_spec, _b_spec, _w_spec, _b_spec],
    out_specs=_row_spec, out_shape=_out_struct)


def kernel(h, block0_edge_index, block1_edge_index, W1, b1, W2, b2, Wp, bp):
    pad_i = jnp.arange(PAD, dtype=jnp.int32)
    pad_dump = (N_N + pad_i % N_DUMP).astype(jnp.int32)
    pad_inb = (pad_i % N_N).astype(jnp.int32)

    def _rows(a, pad):
        a = jnp.concatenate([a.astype(jnp.int32), pad])
        return a.reshape(ROWS_TOTAL, L)

    s0g = _rows(block0_edge_index[0], pad_inb)
    s0d = _rows(block0_edge_index[0], pad_dump)
    d0 = _rows(block0_edge_index[1], pad_dump)
    s1g = _rows(block1_edge_index[0], pad_inb)
    s1d = _rows(block1_edge_index[0], pad_dump)
    d1 = _rows(block1_edge_index[1], pad_dump)

    cnts = _sc_degrees(s0d, d0, s1d, d1).reshape(NC, 4, DEG_N)
    y0 = _tc_first(cnts, h, W1)
    p0 = _sc_message_pass(y0, s0g, d0)
    y1 = _tc_mid(cnts, p0, b1.reshape(1, D), W2)
    p1 = _sc_message_pass(y1, s1g, d1)
    out = _tc_last(cnts, p1, b2.reshape(1, D), Wp, bp.reshape(1, D))
    return out


# speedup vs baseline: 11.5197x; 1.0281x over previous
import functools

import jax
import jax.numpy as jnp
from jax import lax
from jax.experimental import pallas as pl
from jax.experimental.pallas import tpu as pltpu
from jax.experimental.pallas import tpu_sc as plsc

N_N = 10000
N_E = 320000
D = 128
NC = 2
NS = 16
L = 128
N_DUMP = 8
ACC_N = N_N + N_DUMP
ROWS_TOTAL = 2560
PAD = ROWS_TOTAL * L - N_E
ROWS_PER_CORE = ROWS_TOTAL // NC
BULK_ROWS = ROWS_PER_CORE // NS
SLAB = 624
SLAB_REM = N_N - NS * SLAB
DEG_N = 10240
RB = 4096
NB = (N_N + RB - 1) // RB

_mesh = plsc.VectorSubcoreMesh(core_axis_name="c", subcore_axis_name="s")


@functools.partial(
    pl.kernel,
    mesh=_mesh,
    out_type=jax.ShapeDtypeStruct((NC * 4 * DEG_N,), jnp.float32),
    scratch_types=[
        pltpu.VMEM((BULK_ROWS, L), jnp.int32),
        pltpu.VMEM((BULK_ROWS, L), jnp.int32),
        pltpu.VMEM((BULK_ROWS, L), jnp.int32),
        pltpu.VMEM((BULK_ROWS, L), jnp.int32),
        pltpu.VMEM((1, L), jnp.float32),
        pltpu.VMEM((1024,), jnp.float32),
        pltpu.VMEM_SHARED((ACC_N,), jnp.float32),
        pltpu.VMEM_SHARED((ACC_N,), jnp.float32),
        pltpu.VMEM_SHARED((ACC_N,), jnp.float32),
        pltpu.VMEM_SHARED((ACC_N,), jnp.float32),
        pltpu.SemaphoreType.DMA,
        pltpu.SemaphoreType.DMA,
        pltpu.SemaphoreType.DMA,
        pltpu.SemaphoreType.DMA,
    ],
)
def _sc_degrees(s0, d0, s1, d1, out, i0, i1, i2, i3, ones_v, zb_v,
                c0, c1, c2, c3, m0, m1, m2, m3):
    c = lax.axis_index("c")
    s = lax.axis_index("s")
    zeros16 = jnp.zeros((16,), jnp.float32)
    ones16 = jnp.ones((16,), jnp.float32)

    def _fill_z(i, carry):
        zb_v[pl.ds(i * 16, 16)] = zeros16
        return carry
    lax.fori_loop(0, 1024 // 16, _fill_z, 0)

    def _fill_o(i, carry):
        ones_v[0, pl.ds(i * 16, 16)] = ones16
        return carry
    lax.fori_loop(0, L // 16, _fill_o, 0)

    @pl.when(s < 10)
    def _():
        for cref in (c0, c1, c2, c3):
            pltpu.sync_copy(zb_v.at[pl.ds(0, 1000)],
                            cref.at[pl.ds(s * 1000, 1000)])
    plsc.subcore_barrier()

    rbase = c * ROWS_PER_CORE + s * BULK_ROWS
    quad = ((s0, i0, c0, m0), (d0, i1, c1, m1),
            (s1, i2, c2, m2), (d1, i3, c3, m3))
    for arr, iv, cref, m in quad:
        pltpu.sync_copy(arr.at[pl.ds(rbase, BULK_ROWS)], iv)

    for arr, iv, cref, m in quad:
        pltpu.async_copy(ones_v.at[0], cref.at[iv.at[0]], m, add=True)

    def _cnt(j, carry):
        for arr, iv, cref, m in quad:
            pltpu.async_copy(ones_v.at[0], cref.at[iv.at[j + 1]], m, add=True)
        for arr, iv, cref, m in quad:
            pltpu.make_async_copy(ones_v.at[0], cref.at[iv.at[0]], m).wait()
        return carry
    lax.fori_loop(0, BULK_ROWS - 1, _cnt, 0)
    for arr, iv, cref, m in quad:
        pltpu.make_async_copy(ones_v.at[0], cref.at[iv.at[0]], m).wait()

    plsc.subcore_barrier()

    @pl.when(s < 10)
    def _():
        for a, cref in enumerate((c0, c1, c2, c3)):
            pltpu.sync_copy(cref.at[pl.ds(s * 1000, 1000)],
                            zb_v.at[pl.ds(0, 1000)])
            pltpu.sync_copy(
                zb_v.at[pl.ds(0, 1000)],
                out.at[pl.ds((c * 4 + a) * DEG_N + s * 1000, 1000)])


@functools.partial(
    pl.kernel,
    mesh=_mesh,
    out_type=jax.ShapeDtypeStruct((NC, N_N, D), jnp.float32),
    scratch_types=[
        pltpu.VMEM((BULK_ROWS // 2, L), jnp.int32),
        pltpu.VMEM((BULK_ROWS // 2, L), jnp.int32),
        pltpu.VMEM((L, D), jnp.float32),
        pltpu.VMEM((L, D), jnp.float32),
        pltpu.VMEM_SHARED((ACC_N, D), jnp.float32),
        pltpu.SemaphoreType.DMA,
        pltpu.SemaphoreType.DMA,
        pltpu.SemaphoreType.DMA,
        pltpu.SemaphoreType.DMA,
    ],
)
def _sc_message_pass(y, src, dst, out, sidx, didx, rows0, rows1,
                     acc, gsem0, gsem1, ssem0, ssem1):
    c = lax.axis_index("c")
    s = lax.axis_index("s")
    zeros16 = jnp.zeros((16,), jnp.float32)

    def _zo(i, carry):
        def _zi(k, carry2):
            rows0[i, pl.ds(k * 16, 16)] = zeros16
            return carry2
        return lax.fori_loop(0, D // 16, _zi, carry)
    lax.fori_loop(0, L, _zo, 0)

    slab = s * SLAB
    for off, n in ((0, 128), (128, 128), (256, 128), (384, 128), (512, 112)):
        pltpu.sync_copy(rows0.at[pl.ds(0, n)], acc.at[pl.ds(slab + off, n)])

    @pl.when(s == 0)
    def _():
        pltpu.sync_copy(rows0.at[pl.ds(0, SLAB_REM)],
                        acc.at[pl.ds(NS * SLAB, SLAB_REM)])

    plsc.subcore_barrier()

    HALF = BULK_ROWS // 2
    rbase = c * ROWS_PER_CORE + s * BULK_ROWS
    for phase in range(2):
        pbase = rbase + phase * HALF
        pltpu.sync_copy(src.at[pl.ds(pbase, HALF)], sidx)
        pltpu.sync_copy(dst.at[pl.ds(pbase, HALF)], didx)

        pltpu.async_copy(y.at[sidx.at[0]], rows0, gsem0)

        def _step(it, carry):
            j = it * 2
            @pl.when(it > 0)
            def _():
                pltpu.make_async_copy(rows1, acc.at[didx.at[0]], ssem1).wait()
            pltpu.async_copy(y.at[sidx.at[j + 1]], rows1, gsem1)
            pltpu.make_async_copy(y.at[sidx.at[0]], rows0, gsem0).wait()
            pltpu.async_copy(rows0, acc.at[didx.at[j]], ssem0, add=True)
            pltpu.make_async_copy(rows0, acc.at[didx.at[0]], ssem0).wait()

            @pl.when(it < HALF // 2 - 1)
            def _():
                pltpu.async_copy(y.at[sidx.at[j + 2]], rows0, gsem0)
            pltpu.make_async_copy(y.at[sidx.at[0]], rows1, gsem1).wait()
            pltpu.async_copy(rows1, acc.at[didx.at[j + 1]], ssem1, add=True)
            return carry
        lax.fori_loop(0, HALF // 2, _step, 0)
        pltpu.make_async_copy(rows1, acc.at[didx.at[0]], ssem1).wait()

    plsc.subcore_barrier()
    chunks = ((0, 128), (128, 128), (256, 128), (384, 128), (512, 112))
    bufs, gs, ss = (rows0, rows1), (gsem0, gsem1), (ssem0, ssem1)

    def _cin(k, b):
        off, n = chunks[k]
        pltpu.async_copy(acc.at[pl.ds(slab + off, n)],
                         bufs[b].at[pl.ds(0, n)], gs[b])

    _cin(0, 0)
    for k in range(5):
        off, n = chunks[k]
        b = k % 2
        pltpu.make_async_copy(acc.at[pl.ds(slab + off, n)],
                              bufs[b].at[pl.ds(0, n)], gs[b]).wait()
        if k + 1 < 5:
            nb = (k + 1) % 2
            if k + 1 >= 2:
                poff, pn = chunks[k - 1]
                pltpu.make_async_copy(
                    bufs[nb].at[pl.ds(0, pn)],
                    out.at[c, pl.ds(slab + poff, pn)], ss[nb]).wait()
            _cin(k + 1, nb)
        pltpu.async_copy(bufs[b].at[pl.ds(0, n)],
                         out.at[c, pl.ds(slab + off, n)], ss[b])
    for k in (3, 4):
        off, n = chunks[k]
        pltpu.make_async_copy(bufs[k % 2].at[pl.ds(0, n)],
                              out.at[c, pl.ds(slab + off, n)],
                              ss[k % 2]).wait()

    @pl.when(s == 0)
    def _():
        pltpu.sync_copy(acc.at[pl.ds(NS * SLAB, SLAB_REM)],
                        rows0.at[pl.ds(0, SLAB_REM)])
        pltpu.sync_copy(rows0.at[pl.ds(0, SLAB_REM)],
                        out.at[c, pl.ds(NS * SLAB, SLAB_REM)])


def _tc_first_body(cnt_ref, h_ref, w_ref, o_ref):
    i = pl.program_id(0)
    sl = pl.ds(i * RB, RB)
    deg = cnt_ref[0, 0, sl] + cnt_ref[1, 0, sl]
    ns = lax.rsqrt(jnp.maximum(deg, 1.0))
    o_ref[...] = jnp.dot(h_ref[...] * ns[:, None], w_ref[...],
                         preferred_element_type=jnp.float32)


def _tc_mid_body(cnt_ref, p_ref, b_ref, w_ref, o_ref):
    i = pl.program_id(0)
    sl = pl.ds(i * RB, RB)
    din = cnt_ref[0, 1, sl] + cnt_ref[1, 1, sl]
    dout = cnt_ref[0, 2, sl] + cnt_ref[1, 2, sl]
    nd = lax.rsqrt(jnp.maximum(din, 1.0))
    ns = lax.rsqrt(jnp.maximum(dout, 1.0))
    agg = (p_ref[0, :, :] + p_ref[1, :, :]) * nd[:, None] + b_ref[...]
    o_ref[...] = jnp.dot(agg * ns[:, None], w_ref[...],
                         preferred_element_type=jnp.float32)


def _tc_last_body(cnt_ref, p_ref, b_ref, w_ref, bp_ref, o_ref):
    i = pl.program_id(0)
    sl = pl.ds(i * RB, RB)
    din = cnt_ref[0, 3, sl] + cnt_ref[1, 3, sl]
    nd = lax.rsqrt(jnp.maximum(din, 1.0))
    agg = (p_ref[0, :, :] + p_ref[1, :, :]) * nd[:, None] + b_ref[...]
    o_ref[...] = jnp.dot(agg, w_ref[...],
                         preferred_element_type=jnp.float32) + bp_ref[...]


_cnt_spec = pl.BlockSpec((NC, 4, DEG_N), lambda i: (0, 0, 0))
_row_spec = pl.BlockSpec((RB, D), lambda i: (i, 0))
_p_spec = pl.BlockSpec((NC, RB, D), lambda i: (0, i, 0))
_w_spec = pl.BlockSpec((D, D), lambda i: (0, 0))
_b_spec = pl.BlockSpec((1, D), lambda i: (0, 0))
_out_struct = jax.ShapeDtypeStruct((N_N, D), jnp.float32)

_tc_first = pl.pallas_call(
    _tc_first_body, grid=(NB,),
    in_specs=[_cnt_spec, _row_spec, _w_spec],
    out_specs=_row_spec, out_shape=_out_struct)

_tc_mid = pl.pallas_call(
    _tc_mid_body, grid=(NB,),
    in_specs=[_cnt_spec, _p_spec, _b_spec, _w_spec],
    out_specs=_row_spec, out_shape=_out_struct)

_tc_last = pl.pallas_call(
    _tc_last_body, grid=(NB,),
    in_specs=[_cnt_spec, _p_spec, _b_spec, _w_spec, _b_spec],
    out_specs=_row_spec, out_shape=_out_struct)


def kernel(h, block0_edge_index, block1_edge_index, W1, b1, W2, b2, Wp, bp):
    pad_i = jnp.arange(PAD, dtype=jnp.int32)
    pad_dump = (N_N + pad_i % N_DUMP).astype(jnp.int32)
    pad_inb = (pad_i % N_N).astype(jnp.int32)

    def _rows(a, pad):
        a = jnp.concatenate([a.astype(jnp.int32), pad])
        return a.reshape(ROWS_TOTAL, L)

    s0g = _rows(block0_edge_index[0], pad_inb)
    s0d = _rows(block0_edge_index[0], pad_dump)
    d0 = _rows(block0_edge_index[1], pad_dump)
    s1g = _rows(block1_edge_index[0], pad_inb)
    s1d = _rows(block1_edge_index[0], pad_dump)
    d1 = _rows(block1_edge_index[1], pad_dump)

    cnts = _sc_degrees(s0d, d0, s1d, d1).reshape(NC, 4, DEG_N)
    y0 = _tc_first(cnts, h, W1)
    p0 = _sc_message_pass(y0, s0g, d0)
    y1 = _tc_mid(cnts, p0, b1.reshape(1, D), W2)
    p1 = _sc_message_pass(y1, s1g, d1)
    out = _tc_last(cnts, p1, b2.reshape(1, D), Wp, bp.reshape(1, D))
    return out
